# Initial kernel scaffold; baseline (speedup 1.0000x reference)
import jax
import jax.numpy as jnp
from jax.experimental import pallas as pl


def kernel(feat0, feat1, feat2, labels):
    raise NotImplementedError("write your pallas kernel here")


# speedup vs baseline: 28.5696x; 28.5696x over previous
import jax
import jax.numpy as jnp
from jax.experimental import pallas as pl
from jax.experimental.pallas import tpu as pltpu

_NC = 80
_B = 8
_G = 20
_A = 8576
_BIG_I = 1 << 30
_NK = 10


def _loss_body(f0, f1, f2, lab, out, fs):
    b = pl.program_id(0)
    fs[:, 0:6400] = f0[0]
    fs[:, 6400:8000] = f1[0]
    fs[:, 8064:8464] = f2[0]

    a_i = jax.lax.broadcasted_iota(jnp.int32, (1, _A), 1)
    lvl0 = a_i < 6400
    in01 = a_i < 8064
    valid = lvl0 | ((a_i >= 6400) & (a_i < 8000)) | ((a_i >= 8064) & (a_i < 8464))
    stride = jnp.where(lvl0, 8.0, jnp.where(in01, 16.0, 32.0))
    local = jnp.where(lvl0, a_i, jnp.where(in01, a_i - 6400, a_i - 8064)).astype(jnp.float32)
    wdt = jnp.where(lvl0, 80.0, jnp.where(in01, 40.0, 20.0))
    gy = jnp.floor((local + 0.5) / wdt)
    gx = local - gy * wdt
    xc = (gx + 0.5) * stride
    yc = (gy + 0.5) * stride

    xr = jnp.where(valid, fs[0:1, :], 0.0)
    yr = jnp.where(valid, fs[1:2, :], 0.0)
    wr = jnp.where(valid, fs[2:3, :], 0.0)
    hr = jnp.where(valid, fs[3:4, :], 0.0)
    ob = jnp.where(valid, fs[4:5, :], 0.0)
    cls = jnp.where(valid, fs[5:85, :], 0.0)

    bx = (xr + gx) * stride
    by = (yr + gy) * stride
    bw = jnp.exp(wr) * stride
    bh = jnp.exp(hr) * stride

    so = 1.0 / (1.0 + jnp.exp(-ob))
    sc = 1.0 / (1.0 + jnp.exp(-cls))
    s = jnp.sqrt(sc * so)
    logs = jnp.log(s + 1e-8)
    log1ms = jnp.log(1.0 - s + 1e-8)
    dmat = logs - log1ms
    l0 = jnp.sum(log1ms, axis=0, keepdims=True)
    sbce = jnp.sum(jnp.maximum(cls, 0.0) + jnp.log1p(jnp.exp(-jnp.abs(cls))),
                   axis=0, keepdims=True)

    labv = lab[0]
    gtx = labv[:, 0:1]
    gty = labv[:, 1:2]
    gtw = labv[:, 2:3]
    gth = labv[:, 3:4]
    gcls = labv[:, 4:5]

    in_boxes = ((xc > gtx - 0.5 * gtw) & (xc < gtx + 0.5 * gtw)
                & (yc > gty - 0.5 * gth) & (yc < gty + 0.5 * gth))
    in_centers = ((xc > gtx - 2.5 * stride) & (xc < gtx + 2.5 * stride)
                  & (yc > gty - 2.5 * stride) & (yc < gty + 2.5 * stride))
    fg = (jnp.max((in_boxes | in_centers).astype(jnp.float32), axis=0,
                  keepdims=True) > 0.0) & valid
    in_both = in_boxes & in_centers

    tlx = jnp.maximum(gtx - 0.5 * gtw, bx - 0.5 * bw)
    tly = jnp.maximum(gty - 0.5 * gth, by - 0.5 * bh)
    brx = jnp.minimum(gtx + 0.5 * gtw, bx + 0.5 * bw)
    bry = jnp.minimum(gty + 0.5 * gth, by + 0.5 * bh)
    en = ((tlx < brx) & (tly < bry)).astype(jnp.float32)
    area_i = (brx - tlx) * (bry - tly) * en
    ious = area_i / (gtw * gth + bw * bh - area_i + 1e-16)

    iou_cost = -jnp.log(ious + 1e-8)
    c_i = jax.lax.broadcasted_iota(jnp.int32, (_G, _NC), 1)
    onehot = (gcls.astype(jnp.int32) == c_i).astype(jnp.float32)
    dsel = jax.lax.dot_general(onehot, dmat, (((1,), (0,)), ((), ())),
                               precision=jax.lax.Precision.HIGHEST,
                               preferred_element_type=jnp.float32)
    cpsel = jax.lax.dot_general(onehot, cls, (((1,), (0,)), ((), ())),
                                precision=jax.lax.Precision.HIGHEST,
                                preferred_element_type=jnp.float32)
    cls_cost = -(l0 + dsel)
    cost = (cls_cost + 3.0 * iou_cost
            + 100000.0 * (1.0 - in_both.astype(jnp.float32))
            + 100000.0 * (1.0 - fg.astype(jnp.float32)))
    cost = jnp.where(valid, cost, 1e30)

    cur = jnp.where(fg, ious, 0.0)
    ksum = jnp.zeros((_G, 1), jnp.float32)
    for _ in range(_NK):
        m = jnp.max(cur, axis=1, keepdims=True)
        ksum = ksum + m
        idx = jnp.min(jnp.where(cur == m, a_i, _BIG_I), axis=1, keepdims=True)
        cur = jnp.where(a_i == idx, -1.0, cur)
    dyn_k = jnp.clip(ksum.astype(jnp.int32), 1, _NK)

    curc = cost
    matchf = jnp.zeros((_G, _A), jnp.float32)
    for j in range(_NK):
        m = jnp.min(curc, axis=1, keepdims=True)
        idx = jnp.min(jnp.where(curc == m, a_i, _BIG_I), axis=1, keepdims=True)
        hit = a_i == idx
        matchf = jnp.maximum(matchf, jnp.where(hit & (dyn_k > j), 1.0, 0.0))
        curc = jnp.where(hit, 1e35, curc)

    amg = jnp.sum(matchf, axis=0, keepdims=True)
    minc = jnp.min(cost, axis=0, keepdims=True)
    g_i = jax.lax.broadcasted_iota(jnp.int32, (_G, _A), 0)
    garg = jnp.min(jnp.where(cost == minc, g_i, 99), axis=0, keepdims=True)
    matchf = jnp.where(amg > 1.0, (g_i == garg).astype(jnp.float32), matchf)

    fgf = jnp.max(matchf, axis=0, keepdims=True)
    pious = jnp.sum(matchf * ious, axis=0, keepdims=True)
    tx = jnp.sum(matchf * gtx, axis=0, keepdims=True)
    ty = jnp.sum(matchf * gty, axis=0, keepdims=True)
    tw = jnp.sum(matchf * gtw, axis=0, keepdims=True)
    th = jnp.sum(matchf * gth, axis=0, keepdims=True)
    csel = jnp.sum(matchf * cpsel, axis=0, keepdims=True)

    tlx2 = jnp.maximum(bx - 0.5 * bw, tx - 0.5 * tw)
    tly2 = jnp.maximum(by - 0.5 * bh, ty - 0.5 * th)
    brx2 = jnp.minimum(bx + 0.5 * bw, tx + 0.5 * tw)
    bry2 = jnp.minimum(by + 0.5 * bh, ty + 0.5 * th)
    en2 = ((tlx2 < brx2) & (tly2 < bry2)).astype(jnp.float32)
    ai2 = (brx2 - tlx2) * (bry2 - tly2) * en2
    iou2 = ai2 / (bw * bh + tw * th - ai2 + 1e-16)
    t_iou = jnp.sum((1.0 - iou2 * iou2) * fgf)

    bce_obj = jnp.maximum(ob, 0.0) - ob * fgf + jnp.log1p(jnp.exp(-jnp.abs(ob)))
    t_obj = jnp.sum(jnp.where(valid, bce_obj, 0.0))
    t_cls = jnp.sum(fgf * (sbce - csel * pious))
    t_fg = jnp.sum(fgf)

    li = jax.lax.broadcasted_iota(jnp.int32, (1, 8), 1)
    vec = (jnp.where(li == 0, t_iou, 0.0) + jnp.where(li == 1, t_obj, 0.0)
           + jnp.where(li == 2, t_cls, 0.0) + jnp.where(li == 3, t_fg, 0.0))

    @pl.when(b == 0)
    def _():
        out[...] = vec

    @pl.when(b > 0)
    def _():
        out[...] = out[...] + vec

    @pl.when(b == _B - 1)
    def _():
        acc = out[...]
        num_fg = jnp.maximum(jnp.sum(jnp.where(li == 3, acc, 0.0)), 1.0)
        total = (5.0 * jnp.sum(jnp.where(li == 0, acc, 0.0))
                 + jnp.sum(jnp.where(li == 1, acc, 0.0))
                 + jnp.sum(jnp.where(li == 2, acc, 0.0)))
        out[...] = jnp.where(li == 4, total / num_fg, acc)


def kernel(feat0, feat1, feat2, labels):
    f0 = feat0.reshape(_B, 85, 6400)
    f1 = feat1.reshape(_B, 85, 1600)
    f2 = feat2.reshape(_B, 85, 400)
    out = pl.pallas_call(
        _loss_body,
        grid=(_B,),
        in_specs=[
            pl.BlockSpec((1, 85, 6400), lambda b: (b, 0, 0)),
            pl.BlockSpec((1, 85, 1600), lambda b: (b, 0, 0)),
            pl.BlockSpec((1, 85, 400), lambda b: (b, 0, 0)),
            pl.BlockSpec((1, _G, 5), lambda b: (b, 0, 0)),
        ],
        out_specs=pl.BlockSpec((1, 8), lambda b: (0, 0)),
        out_shape=jax.ShapeDtypeStruct((1, 8), jnp.float32),
        scratch_shapes=[pltpu.VMEM((85, _A), jnp.float32)],
    )(f0, f1, f2, labels)
    return out[0, 4]
